# baseline (speedup 1.0000x reference)
import functools

import jax
import jax.numpy as jnp
from jax.experimental import pallas as pl
from jax.experimental.pallas import tpu as pltpu

HIDDEN = 2048
NUM_EXPERTS = 64
TOP_K = 8
TOKENS = 16384

TM = 512


def _router_body(x_ref, w_ref, scores_ref, wts_ref, idx_ref):
    x = x_ref[...]
    w = w_ref[...]
    logits = jax.lax.dot_general(
        x, w, (((1,), (1,)), ((), ())),
        preferred_element_type=jnp.float32)

    m = jnp.max(logits, axis=-1, keepdims=True)
    e = jnp.exp(logits - m)
    denom = jnp.sum(e, axis=-1, keepdims=True)
    scores = e / denom
    scores_ref[...] = scores

    cols = jax.lax.broadcasted_iota(jnp.int32, scores.shape, 1)
    work = scores
    wts_cols = []
    idx_cols = []
    for _ in range(TOP_K):
        mk = jnp.max(work, axis=-1, keepdims=True)
        amax = jnp.min(jnp.where(work >= mk, cols, NUM_EXPERTS),
                       axis=-1, keepdims=True)
        wts_cols.append(mk)
        idx_cols.append(amax)
        work = jnp.where(cols == amax, -jnp.inf, work)
    wts_ref[...] = jnp.concatenate(wts_cols, axis=1)
    idx_ref[...] = jnp.concatenate(idx_cols, axis=1)


@functools.partial(jax.jit, static_argnames=("interpret",))
def kernel(x, W, interpret=False):
    n_tokens = x.shape[0]
    grid = (n_tokens // TM,)
    return pl.pallas_call(
        _router_body,
        grid=grid,
        in_specs=[
            pl.BlockSpec((TM, HIDDEN), lambda i: (i, 0)),
            pl.BlockSpec((NUM_EXPERTS, HIDDEN), lambda i: (0, 0)),
        ],
        out_specs=[
            pl.BlockSpec((TM, NUM_EXPERTS), lambda i: (i, 0)),
            pl.BlockSpec((TM, TOP_K), lambda i: (i, 0)),
            pl.BlockSpec((TM, TOP_K), lambda i: (i, 0)),
        ],
        out_shape=[
            jax.ShapeDtypeStruct((n_tokens, NUM_EXPERTS), jnp.float32),
            jax.ShapeDtypeStruct((n_tokens, TOP_K), jnp.float32),
            jax.ShapeDtypeStruct((n_tokens, TOP_K), jnp.int32),
        ],
        compiler_params=pltpu.CompilerParams(
            dimension_semantics=("arbitrary",),
        ),
        interpret=interpret,
    )(x, W)


# speedup vs baseline: 1.1752x; 1.1752x over previous
import functools

import jax
import jax.numpy as jnp
from jax.experimental import pallas as pl
from jax.experimental.pallas import tpu as pltpu

HIDDEN = 2048
NUM_EXPERTS = 64
TOP_K = 8
TOKENS = 16384

TM = 1024


def _router_body(x_ref, w_ref, scores_ref, wts_ref, idx_ref):
    x = x_ref[...]
    w = w_ref[...]
    logits = jax.lax.dot_general(
        x, w, (((1,), (1,)), ((), ())),
        preferred_element_type=jnp.float32)

    m = jnp.max(logits, axis=-1, keepdims=True)
    e = jnp.exp(logits - m)
    denom = jnp.sum(e, axis=-1, keepdims=True)
    scores = e / denom
    scores_ref[...] = scores

    cols = jax.lax.broadcasted_iota(jnp.int32, scores.shape, 1)
    work = scores
    wts_cols = []
    idx_cols = []
    for _ in range(TOP_K):
        mk = jnp.max(work, axis=-1, keepdims=True)
        amax = jnp.min(jnp.where(work >= mk, cols, NUM_EXPERTS),
                       axis=-1, keepdims=True)
        wts_cols.append(mk)
        idx_cols.append(amax)
        work = jnp.where(cols == amax, -jnp.inf, work)
    wts_ref[...] = jnp.concatenate(wts_cols, axis=1)
    idx_ref[...] = jnp.concatenate(idx_cols, axis=1)


@functools.partial(jax.jit, static_argnames=("interpret",))
def kernel(x, W, interpret=False):
    n_tokens = x.shape[0]
    grid = (n_tokens // TM,)
    return pl.pallas_call(
        _router_body,
        grid=grid,
        in_specs=[
            pl.BlockSpec((TM, HIDDEN), lambda i: (i, 0)),
            pl.BlockSpec((NUM_EXPERTS, HIDDEN), lambda i: (0, 0)),
        ],
        out_specs=[
            pl.BlockSpec((TM, NUM_EXPERTS), lambda i: (i, 0)),
            pl.BlockSpec((TM, TOP_K), lambda i: (i, 0)),
            pl.BlockSpec((TM, TOP_K), lambda i: (i, 0)),
        ],
        out_shape=[
            jax.ShapeDtypeStruct((n_tokens, NUM_EXPERTS), jnp.float32),
            jax.ShapeDtypeStruct((n_tokens, TOP_K), jnp.float32),
            jax.ShapeDtypeStruct((n_tokens, TOP_K), jnp.int32),
        ],
        compiler_params=pltpu.CompilerParams(
            dimension_semantics=("arbitrary",),
        ),
        interpret=interpret,
    )(x, W)


# speedup vs baseline: 1.1860x; 1.0092x over previous
import functools

import jax
import jax.numpy as jnp
from jax.experimental import pallas as pl
from jax.experimental.pallas import tpu as pltpu

HIDDEN = 2048
NUM_EXPERTS = 64
TOP_K = 8
TOKENS = 16384

TM = 2048


def _router_body(x_ref, w_ref, scores_ref, wts_ref, idx_ref):
    x = x_ref[...]
    w = w_ref[...]
    logits = jax.lax.dot_general(
        x, w, (((1,), (1,)), ((), ())),
        preferred_element_type=jnp.float32)

    m = jnp.max(logits, axis=-1, keepdims=True)
    e = jnp.exp(logits - m)
    denom = jnp.sum(e, axis=-1, keepdims=True)
    scores = e / denom
    scores_ref[...] = scores

    cols = jax.lax.broadcasted_iota(jnp.int32, scores.shape, 1)
    work = scores
    wts_cols = []
    idx_cols = []
    for _ in range(TOP_K):
        mk = jnp.max(work, axis=-1, keepdims=True)
        amax = jnp.min(jnp.where(work >= mk, cols, NUM_EXPERTS),
                       axis=-1, keepdims=True)
        wts_cols.append(mk)
        idx_cols.append(amax)
        work = jnp.where(cols == amax, -jnp.inf, work)
    wts_ref[...] = jnp.concatenate(wts_cols, axis=1)
    idx_ref[...] = jnp.concatenate(idx_cols, axis=1)


@functools.partial(jax.jit, static_argnames=("interpret",))
def kernel(x, W, interpret=False):
    n_tokens = x.shape[0]
    grid = (n_tokens // TM,)
    return pl.pallas_call(
        _router_body,
        grid=grid,
        in_specs=[
            pl.BlockSpec((TM, HIDDEN), lambda i: (i, 0)),
            pl.BlockSpec((NUM_EXPERTS, HIDDEN), lambda i: (0, 0)),
        ],
        out_specs=[
            pl.BlockSpec((TM, NUM_EXPERTS), lambda i: (i, 0)),
            pl.BlockSpec((TM, TOP_K), lambda i: (i, 0)),
            pl.BlockSpec((TM, TOP_K), lambda i: (i, 0)),
        ],
        out_shape=[
            jax.ShapeDtypeStruct((n_tokens, NUM_EXPERTS), jnp.float32),
            jax.ShapeDtypeStruct((n_tokens, TOP_K), jnp.float32),
            jax.ShapeDtypeStruct((n_tokens, TOP_K), jnp.int32),
        ],
        compiler_params=pltpu.CompilerParams(
            dimension_semantics=("arbitrary",),
        ),
        interpret=interpret,
    )(x, W)
